# Initial kernel scaffold; baseline (speedup 1.0000x reference)
import jax
import jax.numpy as jnp
from jax.experimental import pallas as pl


def kernel(x, edge_index, edge_attr, W1, bn_gamma, bn_beta, W2, b2, W3, b3, W4, W5, b5, We, be):
    raise NotImplementedError("write your pallas kernel here")


# speedup vs baseline: 3.1567x; 3.1567x over previous
import functools

import jax
import jax.numpy as jnp
from jax import lax
from jax.experimental import pallas as pl
from jax.experimental.pallas import tpu as pltpu
from jax.experimental.pallas import tpu_sc as plsc

NC = 2
NS = 16
NW = NC * NS
CHUNK = 80
SLOTS = 5
SUPER = CHUNK * SLOTS
SCHUNK = 40
SSUPER = SCHUNK * SLOTS


def _sc_gather_body(npad, epw, x_hbm, ii_hbm, jj_hbm, xi_hbm, xd_hbm,
                    idx_v, xi_v, xj_v, sem_idx, sem_g, sem_w):
    cid = lax.axis_index("c")
    sid = lax.axis_index("s")
    base = (sid * NC + cid) * epw
    nsuper = epw // SUPER

    def step(k, carry):
        off = base + k * SUPER
        for s in range(SLOTS):
            sl = pl.ds(off + s * CHUNK, CHUNK)
            pltpu.async_copy(ii_hbm.at[sl], idx_v.at[2 * s], sem_idx)
            pltpu.async_copy(jj_hbm.at[sl], idx_v.at[2 * s + 1], sem_idx)
        for s in range(2 * SLOTS):
            pltpu.make_async_copy(ii_hbm.at[pl.ds(base, CHUNK)],
                                  idx_v.at[0], sem_idx).wait()
        for s in range(SLOTS):
            dst = pl.ds(s * CHUNK, CHUNK)
            pltpu.async_copy(x_hbm.at[idx_v.at[2 * s]],
                             xi_v.at[dst], sem_g)
            pltpu.async_copy(x_hbm.at[idx_v.at[2 * s + 1]],
                             xj_v.at[dst], sem_g)
        for s in range(2 * SLOTS):
            pltpu.make_async_copy(x_hbm.at[idx_v.at[0]],
                                  xi_v.at[pl.ds(0, CHUNK)], sem_g).wait()

        def sub_row(r, c):
            for q in range(8):
                sl = pl.ds(q * 16, 16)
                xj_v[r, sl] = xi_v[r, sl] - xj_v[r, sl]
            return c

        lax.fori_loop(0, SUPER, sub_row, 0)
        for s in range(SLOTS):
            src = pl.ds(s * CHUNK, CHUNK)
            dst = pl.ds(off + s * CHUNK, CHUNK)
            pltpu.async_copy(xi_v.at[src], xi_hbm.at[dst], sem_w)
            pltpu.async_copy(xj_v.at[src], xd_hbm.at[dst], sem_w)
        for s in range(2 * SLOTS):
            pltpu.make_async_copy(xi_v.at[pl.ds(0, CHUNK)],
                                  xi_hbm.at[pl.ds(base, CHUNK)], sem_w).wait()
        return carry

    lax.fori_loop(0, nsuper, step, 0)


def _sc_gather(x, ii, jj):
    npad, d = x.shape
    e = ii.shape[0]
    epw = e // NW
    mesh = plsc.VectorSubcoreMesh(core_axis_name="c", subcore_axis_name="s")
    f = pl.kernel(
        functools.partial(_sc_gather_body, npad, epw),
        out_type=(jax.ShapeDtypeStruct((e, d), jnp.float32),
                  jax.ShapeDtypeStruct((e, d), jnp.float32)),
        mesh=mesh,
        scratch_types=[
            pltpu.VMEM((2 * SLOTS, CHUNK), jnp.int32),
            pltpu.VMEM((SUPER, d), jnp.float32),
            pltpu.VMEM((SUPER, d), jnp.float32),
            pltpu.SemaphoreType.DMA,
            pltpu.SemaphoreType.DMA,
            pltpu.SemaphoreType.DMA,
        ],
    )
    return f(x, ii, jj)


def _sc_scatter_body(npad, epw, xd_hbm, t_hbm, ii_hbm, part_hbm,
                     accum_sh, i0_v, i1_v, i2_v, i3_v, i4_v, t_v, xd_v,
                     sem_in, sem_add):
    idx_refs = (i0_v, i1_v, i2_v, i3_v, i4_v)
    cid = lax.axis_index("c")
    sid = lax.axis_index("s")
    base = (sid * NC + cid) * epw
    nsuper = epw // SSUPER
    rpt = npad // NS

    def zfill(k, c):
        for q in range(8):
            xd_v[k, pl.ds(q * 16, 16)] = jnp.zeros((16,), jnp.float32)
        return c

    lax.fori_loop(0, 128, zfill, 0)
    row0 = sid * rpt
    for z in range(rpt // 128):
        pltpu.sync_copy(xd_v.at[pl.ds(0, 128)],
                        accum_sh.at[pl.ds(row0 + z * 128, 128)])
    plsc.subcore_barrier()

    lo = jnp.full((16,), -100.0, jnp.float32)
    hi = jnp.full((16,), 100.0, jnp.float32)

    def step(k, carry):
        off = base + k * SSUPER
        for s in range(SLOTS):
            sl = pl.ds(off + s * SCHUNK, SCHUNK)
            pltpu.async_copy(ii_hbm.at[sl], idx_refs[s], sem_in)
            pltpu.async_copy(t_hbm.at[sl], t_v.at[s], sem_in)
            pltpu.async_copy(xd_hbm.at[sl], xd_v.at[pl.ds(s * SCHUNK, SCHUNK)],
                             sem_in)
        for s in range(SLOTS):
            pltpu.make_async_copy(ii_hbm.at[pl.ds(base, SCHUNK)],
                                  idx_refs[0], sem_in).wait()
            pltpu.make_async_copy(t_hbm.at[pl.ds(base, SCHUNK)],
                                  t_v.at[0], sem_in).wait()
            pltpu.make_async_copy(xd_hbm.at[pl.ds(base, SCHUNK)],
                                  xd_v.at[pl.ds(0, SCHUNK)], sem_in).wait()

        def trans_row(s, r, tv):
            for q in range(8):
                sl = pl.ds(q * 16, 16)
                v = xd_v[s * SCHUNK + r, sl] * tv
                xd_v[s * SCHUNK + r, sl] = jnp.minimum(jnp.maximum(v, lo), hi)

        def trans_group(g, c):
            for s in range(SLOTS):
                tv16 = t_v[s, pl.ds(g * 16, 16)]
                for e2 in range(16):
                    trans_row(s, g * 16 + e2, jnp.broadcast_to(tv16[e2], (16,)))
            return c

        lax.fori_loop(0, SCHUNK // 16, trans_group, 0)
        if SCHUNK % 16:
            for s in range(SLOTS):
                tv16 = t_v[s, pl.ds(SCHUNK - 16, 16)]
                for e2 in range(16 - SCHUNK % 16, 16):
                    trans_row(s, SCHUNK - 16 + e2,
                              jnp.broadcast_to(tv16[e2], (16,)))
        for s in range(SLOTS):
            pltpu.sync_copy(xd_v.at[pl.ds(s * SCHUNK, SCHUNK)],
                            accum_sh.at[idx_refs[s]], add=True)
        return carry

    lax.fori_loop(0, nsuper, step, 0)
    plsc.subcore_barrier()

    for z in range(rpt // 128):
        r0 = row0 + z * 128
        pltpu.sync_copy(accum_sh.at[pl.ds(r0, 128)], xd_v.at[pl.ds(0, 128)])
        pltpu.sync_copy(xd_v.at[pl.ds(0, 128)], part_hbm.at[cid, pl.ds(r0, 128)])


def _sc_scatter(xd, t, ii, npad):
    e, d = xd.shape
    epw = e // NW
    mesh = plsc.VectorSubcoreMesh(core_axis_name="c", subcore_axis_name="s")
    f = pl.kernel(
        functools.partial(_sc_scatter_body, npad, epw),
        out_type=jax.ShapeDtypeStruct((NC, npad, d), jnp.float32),
        mesh=mesh,
        scratch_types=(
            [pltpu.VMEM_SHARED((npad, d), jnp.float32)]
            + [pltpu.VMEM((SCHUNK,), jnp.int32) for _ in range(SLOTS)]
            + [pltpu.VMEM((SLOTS, SCHUNK), jnp.float32),
               pltpu.VMEM((SSUPER, d), jnp.float32),
               pltpu.SemaphoreType.DMA,
               pltpu.SemaphoreType.DMA]
        ),
    )
    return f(xd, t, ii)


def _psi(p):
    return jnp.sign(p) * jnp.log(jnp.abs(p) + 1.0)


def _tc_a_body(nsteps, e_total,
               xi_ref, xd_ref, ea_ref, w1ab_ref, w1b_ref, w1c_ref, wnd_ref,
               h_ref, stat_ref, acc_ref):
    pid = pl.program_id(0)

    @pl.when(pid == 0)
    def _():
        acc_ref[...] = jnp.zeros_like(acc_ref)

    xi = xi_ref[...]
    xd = xd_ref[...]
    xj = xi - xd
    norms = _psi(2.0 * xd[:, 0] ** 2 - jnp.sum(xd * xd, axis=1))
    dots = _psi(2.0 * xi[:, 0] * xj[:, 0] - jnp.sum(xi * xj, axis=1))
    h = (jnp.dot(xi, w1ab_ref[...], preferred_element_type=jnp.float32)
         - jnp.dot(xd, w1b_ref[...], preferred_element_type=jnp.float32)
         + jnp.dot(ea_ref[...], w1c_ref[...], preferred_element_type=jnp.float32)
         + norms[:, None] * wnd_ref[0][None, :]
         + dots[:, None] * wnd_ref[1][None, :])
    h_ref[...] = h.astype(jnp.bfloat16)
    acc_ref[0, :] += jnp.sum(h, axis=0)
    acc_ref[1, :] += jnp.sum(h * h, axis=0)

    @pl.when(pid == nsteps - 1)
    def _():
        mu = acc_ref[0, :] / e_total
        var = acc_ref[1, :] / e_total - mu * mu
        stat_ref[0, :] = mu
        stat_ref[1, :] = lax.rsqrt(var + 1e-5)


def _tc_a(xi, xd, ea, w1ab, w1b, w1c, wnd, block):
    e, d = xi.shape
    nsteps = e // block
    return pl.pallas_call(
        functools.partial(_tc_a_body, nsteps, float(e)),
        grid=(nsteps,),
        in_specs=[
            pl.BlockSpec((block, d), lambda i: (i, 0)),
            pl.BlockSpec((block, d), lambda i: (i, 0)),
            pl.BlockSpec((block, 4), lambda i: (i, 0)),
            pl.BlockSpec((d, d), lambda i: (0, 0)),
            pl.BlockSpec((d, d), lambda i: (0, 0)),
            pl.BlockSpec((4, d), lambda i: (0, 0)),
            pl.BlockSpec((2, d), lambda i: (0, 0)),
        ],
        out_specs=[
            pl.BlockSpec((block, d), lambda i: (i, 0)),
            pl.BlockSpec((2, d), lambda i: (0, 0)),
        ],
        out_shape=[
            jax.ShapeDtypeStruct((e, d), jnp.bfloat16),
            jax.ShapeDtypeStruct((2, d), jnp.float32),
        ],
        scratch_shapes=[pltpu.VMEM((2, d), jnp.float32)],
    )(xi, xd, ea, w1ab, w1b, w1c, wnd)


def _tc_b_body(h_ref, stat_ref, gb_ref, w2_ref, b2_ref, w3_ref, b3_ref,
               w45_ref, b5_ref, t_ref):
    h = h_ref[...].astype(jnp.float32)
    mu = stat_ref[0, :]
    inv = stat_ref[1, :]
    hn = (h - mu[None, :]) * inv[None, :] * gb_ref[0][None, :] + gb_ref[1][None, :]
    hn = jnp.maximum(hn, 0.0)
    h2 = jnp.maximum(jnp.dot(hn, w2_ref[...], preferred_element_type=jnp.float32)
                     + b2_ref[0][None, :], 0.0)
    w = jax.nn.sigmoid(jnp.dot(h2, w45_ref[:, 1:2], preferred_element_type=jnp.float32)
                       + b5_ref[0, 0])
    mij = h2 * w
    h3 = jnp.maximum(jnp.dot(mij, w3_ref[...], preferred_element_type=jnp.float32)
                     + b3_ref[0][None, :], 0.0)
    t_ref[...] = jnp.dot(h3, w45_ref[:, 0:1], preferred_element_type=jnp.float32)


def _tc_b(h, stat, gb, w2, b2, w3, b3, w45, b5, block):
    e, d = h.shape
    nsteps = e // block
    return pl.pallas_call(
        _tc_b_body,
        grid=(nsteps,),
        in_specs=[
            pl.BlockSpec((block, d), lambda i: (i, 0)),
            pl.BlockSpec((2, d), lambda i: (0, 0)),
            pl.BlockSpec((2, d), lambda i: (0, 0)),
            pl.BlockSpec((d, d), lambda i: (0, 0)),
            pl.BlockSpec((1, d), lambda i: (0, 0)),
            pl.BlockSpec((d, d), lambda i: (0, 0)),
            pl.BlockSpec((1, d), lambda i: (0, 0)),
            pl.BlockSpec((d, 2), lambda i: (0, 0)),
            pl.BlockSpec((1, 1), lambda i: (0, 0)),
        ],
        out_specs=pl.BlockSpec((block, 1), lambda i: (i, 0)),
        out_shape=jax.ShapeDtypeStruct((e, 1), jnp.float32),
    )(h, stat, gb, w2, b2, w3, b3, w45, b5)


def _tc_update_body(x_ref, p_ref, o_ref):
    o_ref[...] = x_ref[...] + p_ref[0] + p_ref[1]


def _tc_update(x, parts):
    npad, d = x.shape
    rb = 2048
    return pl.pallas_call(
        _tc_update_body,
        grid=(npad // rb,),
        in_specs=[pl.BlockSpec((rb, d), lambda i: (i, 0)),
                  pl.BlockSpec((2, rb, d), lambda i: (0, i, 0))],
        out_specs=pl.BlockSpec((rb, d), lambda i: (i, 0)),
        out_shape=jax.ShapeDtypeStruct((npad, d), jnp.float32),
    )(x, parts)


def _tc_final_body(x_ref, p_ref, we_ref, be_ref, o_ref):
    xn = x_ref[...] + p_ref[0] + p_ref[1]
    o_ref[...] = jax.nn.sigmoid(
        jnp.dot(xn, we_ref[...], preferred_element_type=jnp.float32)
        + be_ref[0][None, :])


def _tc_final(x, parts, we, be, n):
    _, d = x.shape
    out = we.shape[1]
    rb = 2000
    return pl.pallas_call(
        _tc_final_body,
        grid=(n // rb,),
        in_specs=[pl.BlockSpec((rb, d), lambda i: (i, 0)),
                  pl.BlockSpec((2, rb, d), lambda i: (0, i, 0)),
                  pl.BlockSpec((d, out), lambda i: (0, 0)),
                  pl.BlockSpec((1, out), lambda i: (0, 0))],
        out_specs=pl.BlockSpec((rb, out), lambda i: (i, 0)),
        out_shape=jax.ShapeDtypeStruct((n, out), jnp.float32),
    )(x, parts, we, be[None, :])


def kernel(x, edge_index, edge_attr, W1, bn_gamma, bn_beta, W2, b2, W3, b3,
           W4, W5, b5, We, be):
    n, d = x.shape
    e = edge_index.shape[1]
    nlayers = W1.shape[0]
    ii = edge_index[0]
    jj = edge_index[1]
    block = 1600
    npad = ((n + NS * 128 - 1) // (NS * 128)) * NS * 128
    x = jnp.pad(x, ((0, npad - n), (0, 0)))

    for l in range(nlayers):
        w1 = W1[l]
        w1ab = w1[:d] + w1[d:2 * d]
        w1b = w1[d:2 * d]
        w1c = w1[2 * d:2 * d + 4]
        wnd = w1[2 * d + 4:2 * d + 6]
        gb = jnp.stack([bn_gamma[l], bn_beta[l]])
        w45 = jnp.concatenate([W4[l], W5[l]], axis=1)

        xi, xd = _sc_gather(x, ii, jj)
        h, stat = _tc_a(xi, xd, edge_attr, w1ab, w1b, w1c, wnd, block)
        t = _tc_b(h, stat, gb, W2[l], b2[l][None, :], W3[l], b3[l][None, :],
                  w45, b5[l][None, :], block)
        parts = _sc_scatter(xd, t[:, 0], ii, npad)
        if l < nlayers - 1:
            x = _tc_update(x, parts)
        else:
            return _tc_final(x, parts, We, be, n)


# speedup vs baseline: 3.1696x; 1.0041x over previous
import functools

import jax
import jax.numpy as jnp
from jax import lax
from jax.experimental import pallas as pl
from jax.experimental.pallas import tpu as pltpu
from jax.experimental.pallas import tpu_sc as plsc

NC = 2
NS = 16
NW = NC * NS
CHUNK = 80
SLOTS = 5
SUPER = CHUNK * SLOTS
SCHUNK = 40
SSUPER = SCHUNK * SLOTS


def _sc_gather_body(npad, epw, x_hbm, ii_hbm, jj_hbm, xi_hbm, xd_hbm,
                    idx_v, xi_v, xj_v, sem_idx, sem_g, sem_w):
    cid = lax.axis_index("c")
    sid = lax.axis_index("s")
    base = (sid * NC + cid) * epw
    nsuper = epw // SUPER

    def step(k, carry):
        off = base + k * SUPER
        for s in range(SLOTS):
            sl = pl.ds(off + s * CHUNK, CHUNK)
            pltpu.async_copy(ii_hbm.at[sl], idx_v.at[2 * s], sem_idx)
            pltpu.async_copy(jj_hbm.at[sl], idx_v.at[2 * s + 1], sem_idx)
        for s in range(2 * SLOTS):
            pltpu.make_async_copy(ii_hbm.at[pl.ds(base, CHUNK)],
                                  idx_v.at[0], sem_idx).wait()
        for s in range(SLOTS):
            dst = pl.ds(s * CHUNK, CHUNK)
            pltpu.async_copy(x_hbm.at[idx_v.at[2 * s]],
                             xi_v.at[dst], sem_g)
            pltpu.async_copy(x_hbm.at[idx_v.at[2 * s + 1]],
                             xj_v.at[dst], sem_g)
        for s in range(2 * SLOTS):
            pltpu.make_async_copy(x_hbm.at[idx_v.at[0]],
                                  xi_v.at[pl.ds(0, CHUNK)], sem_g).wait()

        def sub_row(r, c):
            for q in range(8):
                sl = pl.ds(q * 16, 16)
                xj_v[r, sl] = xi_v[r, sl] - xj_v[r, sl]
            return c

        lax.fori_loop(0, SUPER, sub_row, 0)
        for s in range(SLOTS):
            src = pl.ds(s * CHUNK, CHUNK)
            dst = pl.ds(off + s * CHUNK, CHUNK)
            pltpu.async_copy(xi_v.at[src], xi_hbm.at[dst], sem_w)
            pltpu.async_copy(xj_v.at[src], xd_hbm.at[dst], sem_w)
        for s in range(2 * SLOTS):
            pltpu.make_async_copy(xi_v.at[pl.ds(0, CHUNK)],
                                  xi_hbm.at[pl.ds(base, CHUNK)], sem_w).wait()
        return carry

    lax.fori_loop(0, nsuper, step, 0)


def _sc_gather(x, ii, jj):
    npad, d = x.shape
    e = ii.shape[0]
    epw = e // NW
    mesh = plsc.VectorSubcoreMesh(core_axis_name="c", subcore_axis_name="s")
    f = pl.kernel(
        functools.partial(_sc_gather_body, npad, epw),
        out_type=(jax.ShapeDtypeStruct((e, d), jnp.float32),
                  jax.ShapeDtypeStruct((e, d), jnp.float32)),
        mesh=mesh,
        scratch_types=[
            pltpu.VMEM((2 * SLOTS, CHUNK), jnp.int32),
            pltpu.VMEM((SUPER, d), jnp.float32),
            pltpu.VMEM((SUPER, d), jnp.float32),
            pltpu.SemaphoreType.DMA,
            pltpu.SemaphoreType.DMA,
            pltpu.SemaphoreType.DMA,
        ],
    )
    return f(x, ii, jj)


def _sc_scatter_body(npad, epw, xd_hbm, t_hbm, ii_hbm, part_hbm,
                     accum_sh, i0_v, i1_v, i2_v, i3_v, i4_v, t_v, xd_v,
                     sem_in, sem_add):
    idx_refs = (i0_v, i1_v, i2_v, i3_v, i4_v)
    cid = lax.axis_index("c")
    sid = lax.axis_index("s")
    base = (sid * NC + cid) * epw
    nsuper = epw // SSUPER
    rpt = npad // NS

    def zfill(k, c):
        for q in range(8):
            xd_v[k, pl.ds(q * 16, 16)] = jnp.zeros((16,), jnp.float32)
        return c

    lax.fori_loop(0, 128, zfill, 0)
    row0 = sid * rpt
    for z in range(rpt // 128):
        pltpu.sync_copy(xd_v.at[pl.ds(0, 128)],
                        accum_sh.at[pl.ds(row0 + z * 128, 128)])
    plsc.subcore_barrier()

    lo = jnp.full((16,), -100.0, jnp.float32)
    hi = jnp.full((16,), 100.0, jnp.float32)

    def step(k, carry):
        off = base + k * SSUPER
        for s in range(SLOTS):
            sl = pl.ds(off + s * SCHUNK, SCHUNK)
            pltpu.async_copy(ii_hbm.at[sl], idx_refs[s], sem_in)
            pltpu.async_copy(t_hbm.at[sl], t_v.at[s], sem_in)
            pltpu.async_copy(xd_hbm.at[sl], xd_v.at[pl.ds(s * SCHUNK, SCHUNK)],
                             sem_in)
        for s in range(SLOTS):
            pltpu.make_async_copy(ii_hbm.at[pl.ds(base, SCHUNK)],
                                  idx_refs[0], sem_in).wait()
            pltpu.make_async_copy(t_hbm.at[pl.ds(base, SCHUNK)],
                                  t_v.at[0], sem_in).wait()
            pltpu.make_async_copy(xd_hbm.at[pl.ds(base, SCHUNK)],
                                  xd_v.at[pl.ds(0, SCHUNK)], sem_in).wait()

        def trans_row(s, r, tv):
            for q in range(8):
                sl = pl.ds(q * 16, 16)
                v = xd_v[s * SCHUNK + r, sl] * tv
                xd_v[s * SCHUNK + r, sl] = jnp.minimum(jnp.maximum(v, lo), hi)

        def trans_group(g, c):
            for s in range(SLOTS):
                tv16 = t_v[s, pl.ds(g * 16, 16)]
                for e2 in range(16):
                    trans_row(s, g * 16 + e2, jnp.broadcast_to(tv16[e2], (16,)))
            return c

        lax.fori_loop(0, SCHUNK // 16, trans_group, 0)
        if SCHUNK % 16:
            for s in range(SLOTS):
                tv16 = t_v[s, pl.ds(SCHUNK - 16, 16)]
                for e2 in range(16 - SCHUNK % 16, 16):
                    trans_row(s, SCHUNK - 16 + e2,
                              jnp.broadcast_to(tv16[e2], (16,)))
        for s in range(SLOTS):
            pltpu.sync_copy(xd_v.at[pl.ds(s * SCHUNK, SCHUNK)],
                            accum_sh.at[idx_refs[s]], add=True)
        return carry

    lax.fori_loop(0, nsuper, step, 0)
    plsc.subcore_barrier()

    for z in range(rpt // 128):
        r0 = row0 + z * 128
        pltpu.sync_copy(accum_sh.at[pl.ds(r0, 128)], xd_v.at[pl.ds(0, 128)])
        pltpu.sync_copy(xd_v.at[pl.ds(0, 128)], part_hbm.at[cid, pl.ds(r0, 128)])


def _sc_scatter(xd, t, ii, npad):
    e, d = xd.shape
    epw = e // NW
    mesh = plsc.VectorSubcoreMesh(core_axis_name="c", subcore_axis_name="s")
    f = pl.kernel(
        functools.partial(_sc_scatter_body, npad, epw),
        out_type=jax.ShapeDtypeStruct((NC, npad, d), jnp.float32),
        mesh=mesh,
        scratch_types=(
            [pltpu.VMEM_SHARED((npad, d), jnp.float32)]
            + [pltpu.VMEM((SCHUNK,), jnp.int32) for _ in range(SLOTS)]
            + [pltpu.VMEM((SLOTS, SCHUNK), jnp.float32),
               pltpu.VMEM((SSUPER, d), jnp.float32),
               pltpu.SemaphoreType.DMA,
               pltpu.SemaphoreType.DMA]
        ),
    )
    return f(xd, t, ii)


def _psi(p):
    return jnp.sign(p) * jnp.log(jnp.abs(p) + 1.0)


def _tc_a_body(nsteps, e_total,
               xi_ref, xd_ref, ea_ref, w1ab_ref, w1b_ref, w1c_ref, wnd_ref,
               h_ref, stat_ref, acc_ref):
    pid = pl.program_id(0)

    @pl.when(pid == 0)
    def _():
        acc_ref[...] = jnp.zeros_like(acc_ref)

    xi = xi_ref[...]
    xd = xd_ref[...]
    xj = xi - xd
    norms = _psi(2.0 * xd[:, 0] ** 2 - jnp.sum(xd * xd, axis=1))
    dots = _psi(2.0 * xi[:, 0] * xj[:, 0] - jnp.sum(xi * xj, axis=1))
    h = (jnp.dot(xi.astype(jnp.bfloat16), w1ab_ref[...],
                 preferred_element_type=jnp.float32)
         - jnp.dot(xd.astype(jnp.bfloat16), w1b_ref[...],
                   preferred_element_type=jnp.float32)
         + jnp.dot(ea_ref[...], w1c_ref[...], preferred_element_type=jnp.float32)
         + norms[:, None] * wnd_ref[0][None, :]
         + dots[:, None] * wnd_ref[1][None, :])
    h_ref[...] = h.astype(jnp.bfloat16)
    acc_ref[0, :] += jnp.sum(h, axis=0)
    acc_ref[1, :] += jnp.sum(h * h, axis=0)

    @pl.when(pid == nsteps - 1)
    def _():
        mu = acc_ref[0, :] / e_total
        var = acc_ref[1, :] / e_total - mu * mu
        stat_ref[0, :] = mu
        stat_ref[1, :] = lax.rsqrt(var + 1e-5)


def _tc_a(xi, xd, ea, w1ab, w1b, w1c, wnd, block):
    e, d = xi.shape
    nsteps = e // block
    return pl.pallas_call(
        functools.partial(_tc_a_body, nsteps, float(e)),
        grid=(nsteps,),
        in_specs=[
            pl.BlockSpec((block, d), lambda i: (i, 0)),
            pl.BlockSpec((block, d), lambda i: (i, 0)),
            pl.BlockSpec((block, 4), lambda i: (i, 0)),
            pl.BlockSpec((d, d), lambda i: (0, 0)),
            pl.BlockSpec((d, d), lambda i: (0, 0)),
            pl.BlockSpec((4, d), lambda i: (0, 0)),
            pl.BlockSpec((2, d), lambda i: (0, 0)),
        ],
        out_specs=[
            pl.BlockSpec((block, d), lambda i: (i, 0)),
            pl.BlockSpec((2, d), lambda i: (0, 0)),
        ],
        out_shape=[
            jax.ShapeDtypeStruct((e, d), jnp.bfloat16),
            jax.ShapeDtypeStruct((2, d), jnp.float32),
        ],
        scratch_shapes=[pltpu.VMEM((2, d), jnp.float32)],
    )(xi, xd, ea, w1ab, w1b, w1c, wnd)


def _tc_b_body(h_ref, stat_ref, gb_ref, w2_ref, b2_ref, w3_ref, b3_ref,
               w45_ref, b5_ref, t_ref):
    h = h_ref[...].astype(jnp.float32)
    mu = stat_ref[0, :]
    inv = stat_ref[1, :]
    hn = (h - mu[None, :]) * inv[None, :] * gb_ref[0][None, :] + gb_ref[1][None, :]
    hn = jnp.maximum(hn, 0.0).astype(jnp.bfloat16)
    h2 = jnp.maximum(jnp.dot(hn, w2_ref[...], preferred_element_type=jnp.float32)
                     + b2_ref[0][None, :], 0.0)
    h2b = h2.astype(jnp.bfloat16)
    w = jax.nn.sigmoid(jnp.dot(h2b, w45_ref[:, 1:2], preferred_element_type=jnp.float32)
                       + b5_ref[0, 0])
    mij = (h2 * w).astype(jnp.bfloat16)
    h3 = jnp.maximum(jnp.dot(mij, w3_ref[...], preferred_element_type=jnp.float32)
                     + b3_ref[0][None, :], 0.0)
    t_ref[...] = jnp.dot(h3.astype(jnp.bfloat16), w45_ref[:, 0:1],
                         preferred_element_type=jnp.float32)


def _tc_b(h, stat, gb, w2, b2, w3, b3, w45, b5, block):
    e, d = h.shape
    nsteps = e // block
    return pl.pallas_call(
        _tc_b_body,
        grid=(nsteps,),
        in_specs=[
            pl.BlockSpec((block, d), lambda i: (i, 0)),
            pl.BlockSpec((2, d), lambda i: (0, 0)),
            pl.BlockSpec((2, d), lambda i: (0, 0)),
            pl.BlockSpec((d, d), lambda i: (0, 0)),
            pl.BlockSpec((1, d), lambda i: (0, 0)),
            pl.BlockSpec((d, d), lambda i: (0, 0)),
            pl.BlockSpec((1, d), lambda i: (0, 0)),
            pl.BlockSpec((d, 2), lambda i: (0, 0)),
            pl.BlockSpec((1, 1), lambda i: (0, 0)),
        ],
        out_specs=pl.BlockSpec((block, 1), lambda i: (i, 0)),
        out_shape=jax.ShapeDtypeStruct((e, 1), jnp.float32),
    )(h, stat, gb, w2, b2, w3, b3, w45, b5)


def _tc_update_body(x_ref, p_ref, o_ref):
    o_ref[...] = x_ref[...] + p_ref[0] + p_ref[1]


def _tc_update(x, parts):
    npad, d = x.shape
    rb = 2048
    return pl.pallas_call(
        _tc_update_body,
        grid=(npad // rb,),
        in_specs=[pl.BlockSpec((rb, d), lambda i: (i, 0)),
                  pl.BlockSpec((2, rb, d), lambda i: (0, i, 0))],
        out_specs=pl.BlockSpec((rb, d), lambda i: (i, 0)),
        out_shape=jax.ShapeDtypeStruct((npad, d), jnp.float32),
    )(x, parts)


def _tc_final_body(x_ref, p_ref, we_ref, be_ref, o_ref):
    xn = x_ref[...] + p_ref[0] + p_ref[1]
    o_ref[...] = jax.nn.sigmoid(
        jnp.dot(xn, we_ref[...], preferred_element_type=jnp.float32)
        + be_ref[0][None, :])


def _tc_final(x, parts, we, be, n):
    _, d = x.shape
    out = we.shape[1]
    rb = 2000
    return pl.pallas_call(
        _tc_final_body,
        grid=(n // rb,),
        in_specs=[pl.BlockSpec((rb, d), lambda i: (i, 0)),
                  pl.BlockSpec((2, rb, d), lambda i: (0, i, 0)),
                  pl.BlockSpec((d, out), lambda i: (0, 0)),
                  pl.BlockSpec((1, out), lambda i: (0, 0))],
        out_specs=pl.BlockSpec((rb, out), lambda i: (i, 0)),
        out_shape=jax.ShapeDtypeStruct((n, out), jnp.float32),
    )(x, parts, we, be[None, :])


def kernel(x, edge_index, edge_attr, W1, bn_gamma, bn_beta, W2, b2, W3, b3,
           W4, W5, b5, We, be):
    n, d = x.shape
    e = edge_index.shape[1]
    nlayers = W1.shape[0]
    ii = edge_index[0]
    jj = edge_index[1]
    block = 1600
    npad = ((n + NS * 128 - 1) // (NS * 128)) * NS * 128
    x = jnp.pad(x, ((0, npad - n), (0, 0)))

    for l in range(nlayers):
        w1 = W1[l]
        w1ab = (w1[:d] + w1[d:2 * d]).astype(jnp.bfloat16)
        w1b = w1[d:2 * d].astype(jnp.bfloat16)
        w1c = w1[2 * d:2 * d + 4]
        wnd = w1[2 * d + 4:2 * d + 6]
        gb = jnp.stack([bn_gamma[l], bn_beta[l]])
        w45 = jnp.concatenate([W4[l], W5[l]], axis=1).astype(jnp.bfloat16)

        xi, xd = _sc_gather(x, ii, jj)
        h, stat = _tc_a(xi, xd, edge_attr, w1ab, w1b, w1c, wnd, block)
        t = _tc_b(h, stat, gb, W2[l].astype(jnp.bfloat16), b2[l][None, :],
                  W3[l].astype(jnp.bfloat16), b3[l][None, :],
                  w45, b5[l][None, :], block)
        parts = _sc_scatter(xd, t[:, 0], ii, npad)
        if l < nlayers - 1:
            x = _tc_update(x, parts)
        else:
            return _tc_final(x, parts, We, be, n)


# speedup vs baseline: 3.3886x; 1.0691x over previous
import functools

import jax
import jax.numpy as jnp
from jax import lax
from jax.experimental import pallas as pl
from jax.experimental.pallas import tpu as pltpu
from jax.experimental.pallas import tpu_sc as plsc

NC = 2
NS = 16
NW = NC * NS
CHUNK = 80
SLOTS = 5
SUPER = CHUNK * SLOTS
SCHUNK = 40
SSUPER = SCHUNK * SLOTS


def _sc_gather_body(npad, epw, x_hbm, ii_hbm, jj_hbm, xi_hbm, xd_hbm,
                    idx_v, xi_v, xj_v, sem_idx, sem_g, sem_w):
    cid = lax.axis_index("c")
    sid = lax.axis_index("s")
    base = (sid * NC + cid) * epw
    nsuper = epw // SUPER

    def step(k, carry):
        off = base + k * SUPER
        for s in range(SLOTS):
            sl = pl.ds(off + s * CHUNK, CHUNK)
            pltpu.async_copy(ii_hbm.at[sl], idx_v.at[2 * s], sem_idx)
            pltpu.async_copy(jj_hbm.at[sl], idx_v.at[2 * s + 1], sem_idx)
        for s in range(2 * SLOTS):
            pltpu.make_async_copy(ii_hbm.at[pl.ds(base, CHUNK)],
                                  idx_v.at[0], sem_idx).wait()
        for s in range(SLOTS):
            dst = pl.ds(s * CHUNK, CHUNK)
            pltpu.async_copy(x_hbm.at[idx_v.at[2 * s]],
                             xi_v.at[dst], sem_g)
            pltpu.async_copy(x_hbm.at[idx_v.at[2 * s + 1]],
                             xj_v.at[dst], sem_g)
        for s in range(2 * SLOTS):
            pltpu.make_async_copy(x_hbm.at[idx_v.at[0]],
                                  xi_v.at[pl.ds(0, CHUNK)], sem_g).wait()

        def sub_row(r, c):
            for q in range(8):
                sl = pl.ds(q * 16, 16)
                xj_v[r, sl] = xi_v[r, sl] - xj_v[r, sl]
            return c

        lax.fori_loop(0, SUPER, sub_row, 0)
        for s in range(SLOTS):
            src = pl.ds(s * CHUNK, CHUNK)
            dst = pl.ds(off + s * CHUNK, CHUNK)
            pltpu.async_copy(xi_v.at[src], xi_hbm.at[dst], sem_w)
            pltpu.async_copy(xj_v.at[src], xd_hbm.at[dst], sem_w)
        for s in range(2 * SLOTS):
            pltpu.make_async_copy(xi_v.at[pl.ds(0, CHUNK)],
                                  xi_hbm.at[pl.ds(base, CHUNK)], sem_w).wait()
        return carry

    lax.fori_loop(0, nsuper, step, 0)


def _sc_gather(x, ii, jj):
    npad, d = x.shape
    e = ii.shape[0]
    epw = e // NW
    mesh = plsc.VectorSubcoreMesh(core_axis_name="c", subcore_axis_name="s")
    f = pl.kernel(
        functools.partial(_sc_gather_body, npad, epw),
        out_type=(jax.ShapeDtypeStruct((e, d), jnp.float32),
                  jax.ShapeDtypeStruct((e, d), jnp.float32)),
        mesh=mesh,
        scratch_types=[
            pltpu.VMEM((2 * SLOTS, CHUNK), jnp.int32),
            pltpu.VMEM((SUPER, d), jnp.float32),
            pltpu.VMEM((SUPER, d), jnp.float32),
            pltpu.SemaphoreType.DMA,
            pltpu.SemaphoreType.DMA,
            pltpu.SemaphoreType.DMA,
        ],
    )
    return f(x, ii, jj)


def _sc_scatter_body(npad, epw, xd_hbm, t_hbm, ii_hbm, part_hbm,
                     accum_sh, i0_v, i1_v, i2_v, i3_v, i4_v, t_v, xd_v,
                     sem_in, sem_add):
    idx_refs = (i0_v, i1_v, i2_v, i3_v, i4_v)
    cid = lax.axis_index("c")
    sid = lax.axis_index("s")
    base = (sid * NC + cid) * epw
    nsuper = epw // SSUPER
    rpt = npad // NS

    def zfill(k, c):
        for q in range(8):
            xd_v[k, pl.ds(q * 16, 16)] = jnp.zeros((16,), jnp.float32)
        return c

    lax.fori_loop(0, 128, zfill, 0)
    row0 = sid * rpt
    for z in range(rpt // 128):
        pltpu.sync_copy(xd_v.at[pl.ds(0, 128)],
                        accum_sh.at[pl.ds(row0 + z * 128, 128)])
    plsc.subcore_barrier()

    lo = jnp.full((16,), -100.0, jnp.float32)
    hi = jnp.full((16,), 100.0, jnp.float32)

    def step(k, carry):
        off = base + k * SSUPER
        for s in range(SLOTS):
            sl = pl.ds(off + s * SCHUNK, SCHUNK)
            pltpu.async_copy(ii_hbm.at[sl], idx_refs[s], sem_in)
            pltpu.async_copy(t_hbm.at[sl], t_v.at[s], sem_in)
            pltpu.async_copy(xd_hbm.at[sl], xd_v.at[pl.ds(s * SCHUNK, SCHUNK)],
                             sem_in)
        for s in range(SLOTS):
            pltpu.make_async_copy(ii_hbm.at[pl.ds(base, SCHUNK)],
                                  idx_refs[0], sem_in).wait()
            pltpu.make_async_copy(t_hbm.at[pl.ds(base, SCHUNK)],
                                  t_v.at[0], sem_in).wait()
            pltpu.make_async_copy(xd_hbm.at[pl.ds(base, SCHUNK)],
                                  xd_v.at[pl.ds(0, SCHUNK)], sem_in).wait()

        def trans_row(s, r, tv):
            for q in range(8):
                sl = pl.ds(q * 16, 16)
                v = xd_v[s * SCHUNK + r, sl] * tv
                xd_v[s * SCHUNK + r, sl] = jnp.minimum(jnp.maximum(v, lo), hi)

        def trans_group(g, c):
            for s in range(SLOTS):
                tv16 = t_v[s, pl.ds(g * 16, 16)]
                for e2 in range(16):
                    trans_row(s, g * 16 + e2, jnp.broadcast_to(tv16[e2], (16,)))
            return c

        lax.fori_loop(0, SCHUNK // 16, trans_group, 0)
        if SCHUNK % 16:
            for s in range(SLOTS):
                tv16 = t_v[s, pl.ds(SCHUNK - 16, 16)]
                for e2 in range(16 - SCHUNK % 16, 16):
                    trans_row(s, SCHUNK - 16 + e2,
                              jnp.broadcast_to(tv16[e2], (16,)))
        for s in range(SLOTS):
            pltpu.sync_copy(xd_v.at[pl.ds(s * SCHUNK, SCHUNK)],
                            accum_sh.at[idx_refs[s]], add=True)
        return carry

    lax.fori_loop(0, nsuper, step, 0)
    plsc.subcore_barrier()

    for z in range(rpt // 128):
        r0 = row0 + z * 128
        pltpu.sync_copy(accum_sh.at[pl.ds(r0, 128)], xd_v.at[pl.ds(0, 128)])
        pltpu.sync_copy(xd_v.at[pl.ds(0, 128)], part_hbm.at[cid, pl.ds(r0, 128)])


def _sc_scatter(xd, t, ii, npad):
    e, d = xd.shape
    epw = e // NW
    mesh = plsc.VectorSubcoreMesh(core_axis_name="c", subcore_axis_name="s")
    f = pl.kernel(
        functools.partial(_sc_scatter_body, npad, epw),
        out_type=jax.ShapeDtypeStruct((NC, npad, d), jnp.float32),
        mesh=mesh,
        scratch_types=(
            [pltpu.VMEM_SHARED((npad, d), jnp.float32)]
            + [pltpu.VMEM((SCHUNK,), jnp.int32) for _ in range(SLOTS)]
            + [pltpu.VMEM((SLOTS, SCHUNK), jnp.float32),
               pltpu.VMEM((SSUPER, d), jnp.float32),
               pltpu.SemaphoreType.DMA,
               pltpu.SemaphoreType.DMA]
        ),
    )
    return f(xd, t, ii)


def _psi(p):
    return jnp.sign(p) * jnp.log(jnp.abs(p) + 1.0)


def _tc_a_body(nsteps, e_total,
               xi_ref, xd_ref, ea_ref, w1ab_ref, w1b_ref, w1c_ref, wnd_ref,
               h_ref, stat_ref, acc_ref):
    pid = pl.program_id(0)

    @pl.when(pid == 0)
    def _():
        acc_ref[...] = jnp.zeros_like(acc_ref)

    xi = xi_ref[...]
    xd = xd_ref[...]
    xj = xi - xd
    norms = _psi(2.0 * xd[:, 0] ** 2 - jnp.sum(xd * xd, axis=1))
    dots = _psi(2.0 * xi[:, 0] * xj[:, 0] - jnp.sum(xi * xj, axis=1))
    h = (jnp.dot(xi.astype(jnp.bfloat16), w1ab_ref[...],
                 preferred_element_type=jnp.float32)
         - jnp.dot(xd.astype(jnp.bfloat16), w1b_ref[...],
                   preferred_element_type=jnp.float32)
         + jnp.dot(ea_ref[...], w1c_ref[...], preferred_element_type=jnp.float32)
         + norms[:, None] * wnd_ref[0][None, :]
         + dots[:, None] * wnd_ref[1][None, :])
    h_ref[...] = h.astype(jnp.bfloat16)
    acc_ref[0, :] += jnp.sum(h, axis=0)
    acc_ref[1, :] += jnp.sum(h * h, axis=0)

    @pl.when(pid == nsteps - 1)
    def _():
        mu = acc_ref[0, :] / e_total
        var = acc_ref[1, :] / e_total - mu * mu
        stat_ref[0, :] = mu
        stat_ref[1, :] = lax.rsqrt(var + 1e-5)


def _tc_a(xi, xd, ea, w1ab, w1b, w1c, wnd, block):
    e, d = xi.shape
    nsteps = e // block
    return pl.pallas_call(
        functools.partial(_tc_a_body, nsteps, float(e)),
        grid=(nsteps,),
        in_specs=[
            pl.BlockSpec((block, d), lambda i: (i, 0)),
            pl.BlockSpec((block, d), lambda i: (i, 0)),
            pl.BlockSpec((block, 4), lambda i: (i, 0)),
            pl.BlockSpec((d, d), lambda i: (0, 0)),
            pl.BlockSpec((d, d), lambda i: (0, 0)),
            pl.BlockSpec((4, d), lambda i: (0, 0)),
            pl.BlockSpec((2, d), lambda i: (0, 0)),
        ],
        out_specs=[
            pl.BlockSpec((block, d), lambda i: (i, 0)),
            pl.BlockSpec((2, d), lambda i: (0, 0)),
        ],
        out_shape=[
            jax.ShapeDtypeStruct((e, d), jnp.bfloat16),
            jax.ShapeDtypeStruct((2, d), jnp.float32),
        ],
        scratch_shapes=[pltpu.VMEM((2, d), jnp.float32)],
    )(xi, xd, ea, w1ab, w1b, w1c, wnd)


def _tc_b_body(h_ref, stat_ref, gb_ref, w2_ref, b2_ref, w3_ref, b3_ref,
               w45_ref, b5_ref, t_ref):
    h = h_ref[...].astype(jnp.float32)
    mu = stat_ref[0, :]
    inv = stat_ref[1, :]
    hn = (h - mu[None, :]) * inv[None, :] * gb_ref[0][None, :] + gb_ref[1][None, :]
    hn = jnp.maximum(hn, 0.0).astype(jnp.bfloat16)
    h2 = jnp.maximum(jnp.dot(hn, w2_ref[...], preferred_element_type=jnp.float32)
                     + b2_ref[0][None, :], 0.0)
    w = jax.nn.sigmoid(jnp.sum(h2 * w45_ref[1][None, :], axis=1, keepdims=True)
                       + b5_ref[0, 0])
    mij = (h2 * w).astype(jnp.bfloat16)
    h3 = jnp.maximum(jnp.dot(mij, w3_ref[...], preferred_element_type=jnp.float32)
                     + b3_ref[0][None, :], 0.0)
    t_ref[...] = jnp.sum(h3 * w45_ref[0][None, :], axis=1)[None, None, :]


def _tc_b(h, stat, gb, w2, b2, w3, b3, w45, b5, block):
    e, d = h.shape
    nsteps = e // block
    return pl.pallas_call(
        _tc_b_body,
        grid=(nsteps,),
        in_specs=[
            pl.BlockSpec((block, d), lambda i: (i, 0)),
            pl.BlockSpec((2, d), lambda i: (0, 0)),
            pl.BlockSpec((2, d), lambda i: (0, 0)),
            pl.BlockSpec((d, d), lambda i: (0, 0)),
            pl.BlockSpec((1, d), lambda i: (0, 0)),
            pl.BlockSpec((d, d), lambda i: (0, 0)),
            pl.BlockSpec((1, d), lambda i: (0, 0)),
            pl.BlockSpec((2, d), lambda i: (0, 0)),
            pl.BlockSpec((1, 1), lambda i: (0, 0)),
        ],
        out_specs=pl.BlockSpec((1, 1, block), lambda i: (i, 0, 0)),
        out_shape=jax.ShapeDtypeStruct((nsteps, 1, block), jnp.float32),
    )(h, stat, gb, w2, b2, w3, b3, w45, b5).reshape(e)


def _tc_update_body(x_ref, p_ref, o_ref):
    o_ref[...] = x_ref[...] + p_ref[0] + p_ref[1]


def _tc_update(x, parts):
    npad, d = x.shape
    rb = 2048
    return pl.pallas_call(
        _tc_update_body,
        grid=(npad // rb,),
        in_specs=[pl.BlockSpec((rb, d), lambda i: (i, 0)),
                  pl.BlockSpec((2, rb, d), lambda i: (0, i, 0))],
        out_specs=pl.BlockSpec((rb, d), lambda i: (i, 0)),
        out_shape=jax.ShapeDtypeStruct((npad, d), jnp.float32),
    )(x, parts)


def _tc_final_body(x_ref, p_ref, we_ref, be_ref, o_ref):
    xn = x_ref[...] + p_ref[0] + p_ref[1]
    o_ref[...] = jax.nn.sigmoid(
        jnp.dot(xn, we_ref[...], preferred_element_type=jnp.float32)
        + be_ref[0][None, :])


def _tc_final(x, parts, we, be, n):
    _, d = x.shape
    out = we.shape[1]
    rb = 2000
    return pl.pallas_call(
        _tc_final_body,
        grid=(n // rb,),
        in_specs=[pl.BlockSpec((rb, d), lambda i: (i, 0)),
                  pl.BlockSpec((2, rb, d), lambda i: (0, i, 0)),
                  pl.BlockSpec((d, out), lambda i: (0, 0)),
                  pl.BlockSpec((1, out), lambda i: (0, 0))],
        out_specs=pl.BlockSpec((rb, out), lambda i: (i, 0)),
        out_shape=jax.ShapeDtypeStruct((n, out), jnp.float32),
    )(x, parts, we, be[None, :])


def kernel(x, edge_index, edge_attr, W1, bn_gamma, bn_beta, W2, b2, W3, b3,
           W4, W5, b5, We, be):
    n, d = x.shape
    e = edge_index.shape[1]
    nlayers = W1.shape[0]
    ii = edge_index[0]
    jj = edge_index[1]
    block_a = 3200
    block_b = 6400
    npad = ((n + NS * 128 - 1) // (NS * 128)) * NS * 128
    x = jnp.pad(x, ((0, npad - n), (0, 0)))

    for l in range(nlayers):
        w1 = W1[l]
        w1ab = (w1[:d] + w1[d:2 * d]).astype(jnp.bfloat16)
        w1b = w1[d:2 * d].astype(jnp.bfloat16)
        w1c = w1[2 * d:2 * d + 4]
        wnd = w1[2 * d + 4:2 * d + 6]
        gb = jnp.stack([bn_gamma[l], bn_beta[l]])
        w45 = jnp.stack([W4[l][:, 0], W5[l][:, 0]])

        xi, xd = _sc_gather(x, ii, jj)
        h, stat = _tc_a(xi, xd, edge_attr, w1ab, w1b, w1c, wnd, block_a)
        t = _tc_b(h, stat, gb, W2[l].astype(jnp.bfloat16), b2[l][None, :],
                  W3[l].astype(jnp.bfloat16), b3[l][None, :],
                  w45, b5[l][None, :], block_b)
        parts = _sc_scatter(xd, t, ii, npad)
        if l < nlayers - 1:
            x = _tc_update(x, parts)
        else:
            return _tc_final(x, parts, We, be, n)


# speedup vs baseline: 3.4097x; 1.0062x over previous
import functools

import jax
import jax.numpy as jnp
from jax import lax
from jax.experimental import pallas as pl
from jax.experimental.pallas import tpu as pltpu
from jax.experimental.pallas import tpu_sc as plsc

NC = 2
NS = 16
NW = NC * NS
CHUNK = 80
SLOTS = 5
SUPER = CHUNK * SLOTS
SCHUNK = 40
SSUPER = SCHUNK * SLOTS


def _sc_gather_body(npad, epw, x_hbm, ii_hbm, jj_hbm, xi_hbm, xd_hbm,
                    idx_v, xi_v, xj_v, sem_idx, sem_g, sem_w):
    cid = lax.axis_index("c")
    sid = lax.axis_index("s")
    base = (sid * NC + cid) * epw
    nsuper = epw // SUPER

    def step(k, carry):
        off = base + k * SUPER
        for s in range(SLOTS):
            sl = pl.ds(off + s * CHUNK, CHUNK)
            pltpu.async_copy(ii_hbm.at[sl], idx_v.at[2 * s], sem_idx)
            pltpu.async_copy(jj_hbm.at[sl], idx_v.at[2 * s + 1], sem_idx)
        for s in range(2 * SLOTS):
            pltpu.make_async_copy(ii_hbm.at[pl.ds(base, CHUNK)],
                                  idx_v.at[0], sem_idx).wait()
        for s in range(SLOTS):
            dst = pl.ds(s * CHUNK, CHUNK)
            pltpu.async_copy(x_hbm.at[idx_v.at[2 * s]],
                             xi_v.at[dst], sem_g)
            pltpu.async_copy(x_hbm.at[idx_v.at[2 * s + 1]],
                             xj_v.at[dst], sem_g)
        for s in range(2 * SLOTS):
            pltpu.make_async_copy(x_hbm.at[idx_v.at[0]],
                                  xi_v.at[pl.ds(0, CHUNK)], sem_g).wait()

        def sub_row(r, c):
            for q in range(8):
                sl = pl.ds(q * 16, 16)
                xj_v[r, sl] = xi_v[r, sl] - xj_v[r, sl]
            return c

        lax.fori_loop(0, SUPER, sub_row, 0)
        for s in range(SLOTS):
            src = pl.ds(s * CHUNK, CHUNK)
            dst = pl.ds(off + s * CHUNK, CHUNK)
            pltpu.async_copy(xi_v.at[src], xi_hbm.at[dst], sem_w)
            pltpu.async_copy(xj_v.at[src], xd_hbm.at[dst], sem_w)
        for s in range(2 * SLOTS):
            pltpu.make_async_copy(xi_v.at[pl.ds(0, CHUNK)],
                                  xi_hbm.at[pl.ds(base, CHUNK)], sem_w).wait()
        return carry

    lax.fori_loop(0, nsuper, step, 0)


def _sc_gather(x, ii, jj):
    npad, d = x.shape
    e = ii.shape[0]
    epw = e // NW
    mesh = plsc.VectorSubcoreMesh(core_axis_name="c", subcore_axis_name="s")
    f = pl.kernel(
        functools.partial(_sc_gather_body, npad, epw),
        out_type=(jax.ShapeDtypeStruct((e, d), jnp.float32),
                  jax.ShapeDtypeStruct((e, d), jnp.float32)),
        mesh=mesh,
        scratch_types=[
            pltpu.VMEM((2 * SLOTS, CHUNK), jnp.int32),
            pltpu.VMEM((SUPER, d), jnp.float32),
            pltpu.VMEM((SUPER, d), jnp.float32),
            pltpu.SemaphoreType.DMA,
            pltpu.SemaphoreType.DMA,
            pltpu.SemaphoreType.DMA,
        ],
    )
    return f(x, ii, jj)


def _sc_scatter_body(npad, epw, xd_hbm, t_hbm, ii_hbm, part_hbm,
                     accum_sh, i0_v, i1_v, i2_v, i3_v, i4_v, t_v, xd_v,
                     sem_in, sem_add):
    idx_refs = (i0_v, i1_v, i2_v, i3_v, i4_v)
    cid = lax.axis_index("c")
    sid = lax.axis_index("s")
    base = (sid * NC + cid) * epw
    nsuper = epw // SSUPER
    rpt = npad // NS

    def zfill(k, c):
        for q in range(8):
            xd_v[k, pl.ds(q * 16, 16)] = jnp.zeros((16,), jnp.float32)
        return c

    lax.fori_loop(0, 128, zfill, 0)
    row0 = sid * rpt
    for z in range(rpt // 128):
        pltpu.sync_copy(xd_v.at[pl.ds(0, 128)],
                        accum_sh.at[pl.ds(row0 + z * 128, 128)])
    plsc.subcore_barrier()

    lo = jnp.full((16,), -100.0, jnp.float32)
    hi = jnp.full((16,), 100.0, jnp.float32)

    def step(k, carry):
        off = base + k * SSUPER
        for s in range(SLOTS):
            sl = pl.ds(off + s * SCHUNK, SCHUNK)
            pltpu.async_copy(ii_hbm.at[sl], idx_refs[s], sem_in)
            pltpu.async_copy(t_hbm.at[sl], t_v.at[s], sem_in)
            pltpu.async_copy(xd_hbm.at[sl], xd_v.at[pl.ds(s * SCHUNK, SCHUNK)],
                             sem_in)
        for s in range(SLOTS):
            pltpu.make_async_copy(ii_hbm.at[pl.ds(base, SCHUNK)],
                                  idx_refs[0], sem_in).wait()
            pltpu.make_async_copy(t_hbm.at[pl.ds(base, SCHUNK)],
                                  t_v.at[0], sem_in).wait()
            pltpu.make_async_copy(xd_hbm.at[pl.ds(base, SCHUNK)],
                                  xd_v.at[pl.ds(0, SCHUNK)], sem_in).wait()

        def trans_row(s, r, tv):
            for q in range(8):
                sl = pl.ds(q * 16, 16)
                v = xd_v[s * SCHUNK + r, sl] * tv
                xd_v[s * SCHUNK + r, sl] = jnp.minimum(jnp.maximum(v, lo), hi)

        def trans_group(g, c):
            for s in range(SLOTS):
                tv16 = t_v[s, pl.ds(g * 16, 16)]
                for e2 in range(16):
                    trans_row(s, g * 16 + e2, jnp.broadcast_to(tv16[e2], (16,)))
            return c

        lax.fori_loop(0, SCHUNK // 16, trans_group, 0)
        if SCHUNK % 16:
            for s in range(SLOTS):
                tv16 = t_v[s, pl.ds(SCHUNK - 16, 16)]
                for e2 in range(16 - SCHUNK % 16, 16):
                    trans_row(s, SCHUNK - 16 + e2,
                              jnp.broadcast_to(tv16[e2], (16,)))
        for s in range(SLOTS):
            pltpu.sync_copy(xd_v.at[pl.ds(s * SCHUNK, SCHUNK)],
                            accum_sh.at[idx_refs[s]], add=True)
        return carry

    lax.fori_loop(0, nsuper, step, 0)
    plsc.subcore_barrier()

    for z in range(rpt // 128):
        r0 = row0 + z * 128
        pltpu.sync_copy(accum_sh.at[pl.ds(r0, 128)], xd_v.at[pl.ds(0, 128)])
        pltpu.sync_copy(xd_v.at[pl.ds(0, 128)], part_hbm.at[cid, pl.ds(r0, 128)])


def _sc_scatter(xd, t, ii, npad):
    e, d = xd.shape
    epw = e // NW
    mesh = plsc.VectorSubcoreMesh(core_axis_name="c", subcore_axis_name="s")
    f = pl.kernel(
        functools.partial(_sc_scatter_body, npad, epw),
        out_type=jax.ShapeDtypeStruct((NC, npad, d), jnp.float32),
        mesh=mesh,
        scratch_types=(
            [pltpu.VMEM_SHARED((npad, d), jnp.float32)]
            + [pltpu.VMEM((SCHUNK,), jnp.int32) for _ in range(SLOTS)]
            + [pltpu.VMEM((SLOTS, SCHUNK), jnp.float32),
               pltpu.VMEM((SSUPER, d), jnp.float32),
               pltpu.SemaphoreType.DMA,
               pltpu.SemaphoreType.DMA]
        ),
    )
    return f(xd, t, ii)


def _psi(p):
    return jnp.sign(p) * jnp.log(jnp.abs(p) + 1.0)


def _tc_a_body(nsteps, e_total,
               xi_ref, xd_ref, ea_ref, w1ab_ref, w1b_ref, w1c_ref, wnd_ref,
               h_ref, stat_ref, acc_ref):
    pid = pl.program_id(0)

    @pl.when(pid == 0)
    def _():
        acc_ref[...] = jnp.zeros_like(acc_ref)

    xi = xi_ref[...]
    xd = xd_ref[...]
    xj = xi - xd
    norms = _psi(2.0 * xd[:, 0] ** 2 - jnp.sum(xd * xd, axis=1))
    dots = _psi(2.0 * xi[:, 0] * xj[:, 0] - jnp.sum(xi * xj, axis=1))
    h = (jnp.dot(xi.astype(jnp.bfloat16), w1ab_ref[...],
                 preferred_element_type=jnp.float32)
         - jnp.dot(xd.astype(jnp.bfloat16), w1b_ref[...],
                   preferred_element_type=jnp.float32)
         + jnp.dot(ea_ref[...], w1c_ref[...], preferred_element_type=jnp.float32)
         + norms[:, None] * wnd_ref[0][None, :]
         + dots[:, None] * wnd_ref[1][None, :])
    h_ref[...] = h.astype(jnp.bfloat16)
    acc_ref[0, :] += jnp.sum(h, axis=0)
    acc_ref[1, :] += jnp.sum(h * h, axis=0)

    @pl.when(pid == nsteps - 1)
    def _():
        mu = acc_ref[0, :] / e_total
        var = acc_ref[1, :] / e_total - mu * mu
        stat_ref[0, :] = mu
        stat_ref[1, :] = lax.rsqrt(var + 1e-5)


def _tc_a(xi, xd, ea, w1ab, w1b, w1c, wnd, block):
    e, d = xi.shape
    nsteps = e // block
    return pl.pallas_call(
        functools.partial(_tc_a_body, nsteps, float(e)),
        grid=(nsteps,),
        in_specs=[
            pl.BlockSpec((block, d), lambda i: (i, 0)),
            pl.BlockSpec((block, d), lambda i: (i, 0)),
            pl.BlockSpec((block, 4), lambda i: (i, 0)),
            pl.BlockSpec((d, d), lambda i: (0, 0)),
            pl.BlockSpec((d, d), lambda i: (0, 0)),
            pl.BlockSpec((4, d), lambda i: (0, 0)),
            pl.BlockSpec((2, d), lambda i: (0, 0)),
        ],
        out_specs=[
            pl.BlockSpec((block, d), lambda i: (i, 0)),
            pl.BlockSpec((2, d), lambda i: (0, 0)),
        ],
        out_shape=[
            jax.ShapeDtypeStruct((e, d), jnp.bfloat16),
            jax.ShapeDtypeStruct((2, d), jnp.float32),
        ],
        scratch_shapes=[pltpu.VMEM((2, d), jnp.float32)],
    )(xi, xd, ea, w1ab, w1b, w1c, wnd)


def _tc_b_body(h_ref, stat_ref, gb_ref, w2_ref, b2_ref, w3_ref,
               b3_ref, w45_ref, b5_ref, t_ref):
    scale = stat_ref[1, :] * gb_ref[0]
    shift = gb_ref[1] - stat_ref[0, :] * scale
    h = h_ref[...]
    hn = h * scale.astype(jnp.bfloat16)[None, :] + shift.astype(jnp.bfloat16)[None, :]
    hn = jnp.maximum(hn, jnp.bfloat16(0.0))
    h2 = jnp.maximum(jnp.dot(hn, w2_ref[...], preferred_element_type=jnp.float32)
                     + b2_ref[0][None, :], 0.0).astype(jnp.bfloat16)
    w = jax.nn.sigmoid(jnp.sum((h2 * w45_ref[1][None, :]).astype(jnp.float32),
                               axis=1, keepdims=True) + b5_ref[0, 0])
    mij = h2 * w.astype(jnp.bfloat16)
    h3 = jnp.maximum(jnp.dot(mij, w3_ref[...], preferred_element_type=jnp.float32)
                     + b3_ref[0][None, :], 0.0).astype(jnp.bfloat16)
    t_ref[...] = jnp.sum((h3 * w45_ref[0][None, :]).astype(jnp.float32),
                         axis=1)[None, None, :]


def _tc_b(h, stat, gb, w2, b2, w3, b3, w45, b5, block):
    e, d = h.shape
    nsteps = e // block
    return pl.pallas_call(
        _tc_b_body,
        grid=(nsteps,),
        in_specs=[
            pl.BlockSpec((block, d), lambda i: (i, 0)),
            pl.BlockSpec((2, d), lambda i: (0, 0)),
            pl.BlockSpec((2, d), lambda i: (0, 0)),
            pl.BlockSpec((d, d), lambda i: (0, 0)),
            pl.BlockSpec((1, d), lambda i: (0, 0)),
            pl.BlockSpec((d, d), lambda i: (0, 0)),
            pl.BlockSpec((1, d), lambda i: (0, 0)),
            pl.BlockSpec((2, d), lambda i: (0, 0)),
            pl.BlockSpec((1, 1), lambda i: (0, 0)),
        ],
        out_specs=pl.BlockSpec((1, 1, block), lambda i: (i, 0, 0)),
        out_shape=jax.ShapeDtypeStruct((nsteps, 1, block), jnp.float32),
    )(h, stat, gb, w2, b2, w3, b3, w45, b5).reshape(e)


def _tc_update_body(x_ref, p_ref, o_ref):
    o_ref[...] = x_ref[...] + p_ref[0] + p_ref[1]


def _tc_update(x, parts):
    npad, d = x.shape
    rb = 2048
    return pl.pallas_call(
        _tc_update_body,
        grid=(npad // rb,),
        in_specs=[pl.BlockSpec((rb, d), lambda i: (i, 0)),
                  pl.BlockSpec((2, rb, d), lambda i: (0, i, 0))],
        out_specs=pl.BlockSpec((rb, d), lambda i: (i, 0)),
        out_shape=jax.ShapeDtypeStruct((npad, d), jnp.float32),
    )(x, parts)


def _tc_final_body(x_ref, p_ref, we_ref, be_ref, o_ref):
    xn = x_ref[...] + p_ref[0] + p_ref[1]
    o_ref[...] = jax.nn.sigmoid(
        jnp.dot(xn, we_ref[...], preferred_element_type=jnp.float32)
        + be_ref[0][None, :])


def _tc_final(x, parts, we, be, n):
    _, d = x.shape
    out = we.shape[1]
    rb = 2000
    return pl.pallas_call(
        _tc_final_body,
        grid=(n // rb,),
        in_specs=[pl.BlockSpec((rb, d), lambda i: (i, 0)),
                  pl.BlockSpec((2, rb, d), lambda i: (0, i, 0)),
                  pl.BlockSpec((d, out), lambda i: (0, 0)),
                  pl.BlockSpec((1, out), lambda i: (0, 0))],
        out_specs=pl.BlockSpec((rb, out), lambda i: (i, 0)),
        out_shape=jax.ShapeDtypeStruct((n, out), jnp.float32),
    )(x, parts, we, be[None, :])


def kernel(x, edge_index, edge_attr, W1, bn_gamma, bn_beta, W2, b2, W3, b3,
           W4, W5, b5, We, be):
    n, d = x.shape
    e = edge_index.shape[1]
    nlayers = W1.shape[0]
    ii = edge_index[0]
    jj = edge_index[1]
    block_a = 3200
    block_b = 6400
    npad = ((n + NS * 128 - 1) // (NS * 128)) * NS * 128
    x = jnp.pad(x, ((0, npad - n), (0, 0)))

    for l in range(nlayers):
        w1 = W1[l]
        w1ab = (w1[:d] + w1[d:2 * d]).astype(jnp.bfloat16)
        w1b = w1[d:2 * d].astype(jnp.bfloat16)
        w1c = w1[2 * d:2 * d + 4]
        wnd = w1[2 * d + 4:2 * d + 6]
        gb = jnp.stack([bn_gamma[l], bn_beta[l]])
        w45 = jnp.stack([W4[l][:, 0], W5[l][:, 0]]).astype(jnp.bfloat16)

        xi, xd = _sc_gather(x, ii, jj)
        h, stat = _tc_a(xi, xd, edge_attr, w1ab, w1b, w1c, wnd, block_a)
        t = _tc_b(h, stat, gb, W2[l].astype(jnp.bfloat16), b2[l][None, :],
                  W3[l].astype(jnp.bfloat16), b3[l][None, :],
                  w45, b5[l][None, :], block_b)
        parts = _sc_scatter(xd, t, ii, npad)
        if l < nlayers - 1:
            x = _tc_update(x, parts)
        else:
            return _tc_final(x, parts, We, be, n)
